# Initial kernel scaffold; baseline (speedup 1.0000x reference)
import jax
import jax.numpy as jnp
from jax.experimental import pallas as pl


def kernel(x, adj, wild_adj, wild_feature, nodes, mutaion_site, aux, fc0_w, fc0_b, conv_W, fc_w, fc_b, fc2_w, fc2_b):
    raise NotImplementedError("write your pallas kernel here")


# speedup vs baseline: 1.0791x; 1.0791x over previous
import math

import jax
import jax.numpy as jnp
from jax.experimental import pallas as pl
from jax.experimental.pallas import tpu as pltpu

N = 8192
NFEAT = 128
NHID = 32
NLAYERS = 4
M = 16
LAMDA = 0.5
ALPHA = 0.1

BR = 256
NRB = N // BR

_THETAS = [math.log(LAMDA / l + 1.0) for l in range(1, NLAYERS + 1)]
_A_DT = jnp.bfloat16
_C_DT = jnp.bfloat16


def _body(mut_ref,
          x_ref, wf_ref, adj_ref, wadj_ref, convw_ref, fc0w_ref, fc0b_ref,
          fcwt_ref, fcb_ref, fc2w_ref, fc2b_ref, aux_ref,
          o1_ref, o2_ref,
          h0_s, cur_s, fin_s):
    l = pl.program_id(0)
    rb = pl.program_id(1)
    r0 = rb * BR

    def _store_split(val, parity, b):
        v_hi = val.astype(jnp.bfloat16)
        v_lo = (val - v_hi.astype(jnp.float32)).astype(jnp.bfloat16)
        cur_s[parity, b, pl.ds(r0, BR), 0:NHID] = v_hi
        cur_s[parity, b, pl.ds(r0, BR), NHID:2 * NHID] = v_lo

    def _rows_f32(parity, b):
        blk = cur_s[parity, b, pl.ds(r0, BR), :].astype(jnp.float32)
        return blk[:, :NHID] + blk[:, NHID:]

    @pl.when(l == 0)
    def _prologue():
        for b, ref in ((0, x_ref), (1, wf_ref)):
            h = jnp.maximum(ref[...] @ fc0w_ref[...] + fc0b_ref[...], 0.0)
            h0_s[b, pl.ds(r0, BR), :] = h
            _store_split(h, 0, b)

    @pl.when(l > 0)
    def _layer():
        lm1 = l - 1
        rp = lm1 % 2
        wp = l % 2
        theta = jnp.where(l == 1, _THETAS[0],
                jnp.where(l == 2, _THETAS[1],
                jnp.where(l == 3, _THETAS[2], _THETAS[3]))).astype(jnp.float32)
        w = convw_ref[lm1]
        for b, aref in ((0, adj_ref), (1, wadj_ref)):
            a = aref[...].astype(_A_DT)
            rhs = cur_s[rp, b, :, :]
            hh = jnp.dot(a, rhs, preferred_element_type=jnp.float32)
            hi = hh[:, :NHID] + hh[:, NHID:]
            sup = (1.0 - ALPHA) * hi + ALPHA * h0_s[b, pl.ds(r0, BR), :]
            inp_rows = _rows_f32(rp, b)
            out = theta * jnp.dot(sup, w, preferred_element_type=jnp.float32) \
                + (1.0 - theta) * sup + inp_rows
            new = jnp.maximum(out, 0.0)
            _store_split(new, wp, b)

            @pl.when(l == NLAYERS)
            def _():
                fin_s[b, pl.ds(r0, BR), :] = new

    @pl.when((l == NLAYERS) & (rb == NRB - 1))
    def _head():
        sums = []
        for b in (0, 1):
            acc = jnp.zeros((1, NHID), dtype=jnp.float32)
            for i in range(M):
                acc = acc + fin_s[b, pl.ds(mut_ref[i], 1), :]
            sums.append(acc)
        differ = sums[0] - sums[1]
        lid = jnp.sum(differ * fcwt_ref[...], axis=1, keepdims=True) \
            + fcb_ref[0]
        o2_ref[...] = lid
        relu_lid = jnp.maximum(lid, 0.0)
        o1_ref[...] = relu_lid * fc2w_ref[0] \
            + (aux_ref[4] * M) * fc2w_ref[1] \
            + (aux_ref[5] * M) * fc2w_ref[2] + fc2b_ref[0]


def kernel(x, adj, wild_adj, wild_feature, nodes, mutaion_site, aux,
           fc0_w, fc0_b, conv_W, fc_w, fc_b, fc2_w, fc2_b):
    del nodes
    grid = (NLAYERS + 1, NRB)

    def _vmem(shape, imap):
        return pl.BlockSpec(shape, imap)

    in_specs = [
        _vmem((BR, NFEAT), lambda l, rb, *_: (jnp.where(l == 0, rb, 0), 0)),
        _vmem((BR, NFEAT), lambda l, rb, *_: (jnp.where(l == 0, rb, 0), 0)),
        _vmem((BR, N), lambda l, rb, *_: (jnp.where(l == 0, 0, rb), 0)),
        _vmem((BR, N), lambda l, rb, *_: (jnp.where(l == 0, 0, rb), 0)),
        _vmem((NLAYERS, NHID, NHID), lambda l, rb, *_: (0, 0, 0)),
        _vmem((NFEAT, NHID), lambda l, rb, *_: (0, 0)),
        _vmem((1, NHID), lambda l, rb, *_: (0, 0)),
        _vmem((1, NHID), lambda l, rb, *_: (0, 0)),
        pl.BlockSpec(memory_space=pltpu.SMEM),
        pl.BlockSpec(memory_space=pltpu.SMEM),
        pl.BlockSpec(memory_space=pltpu.SMEM),
        pl.BlockSpec(memory_space=pltpu.SMEM),
    ]
    out_specs = [
        _vmem((1, 1), lambda l, rb, *_: (0, 0)),
        _vmem((1, 1), lambda l, rb, *_: (0, 0)),
    ]
    grid_spec = pltpu.PrefetchScalarGridSpec(
        num_scalar_prefetch=1,
        grid=grid,
        in_specs=in_specs,
        out_specs=out_specs,
        scratch_shapes=[
            pltpu.VMEM((2, N, NHID), jnp.float32),
            pltpu.VMEM((2, 2, N, 2 * NHID), jnp.bfloat16),
            pltpu.VMEM((2, N, NHID), jnp.float32),
        ],
    )
    o1, o2 = pl.pallas_call(
        _body,
        grid_spec=grid_spec,
        out_shape=[
            jax.ShapeDtypeStruct((1, 1), jnp.float32),
            jax.ShapeDtypeStruct((1, 1), jnp.float32),
        ],
        compiler_params=pltpu.CompilerParams(
            dimension_semantics=("arbitrary", "arbitrary"),
        ),
    )(mutaion_site.astype(jnp.int32),
      x, wild_feature, adj, wild_adj, conv_W, fc0_w,
      fc0_b.reshape(1, NHID), fc_w.reshape(1, NHID),
      fc_b, fc2_w.reshape(3), fc2_b, aux)
    return (o1.reshape(1), o2.reshape(1))


# speedup vs baseline: 1.1792x; 1.0928x over previous
import math

import jax
import jax.numpy as jnp
from jax.experimental import pallas as pl
from jax.experimental.pallas import tpu as pltpu

N = 8192
NFEAT = 128
NHID = 32
NLAYERS = 4
M = 16
LAMDA = 0.5
ALPHA = 0.1

BR_A = 256
NRB_A = N // BR_A
BR_B = 256
NRB_B = N // BR_B

_THETAS = [math.log(LAMDA / l + 1.0) for l in range(1, NLAYERS + 1)]


def _split_pair(val):
    v_hi = val.astype(jnp.bfloat16)
    v_lo = (val - v_hi.astype(jnp.float32)).astype(jnp.bfloat16)
    return v_hi, v_lo


def _body_a(x_ref, wf_ref, adj_ref, wadj_ref, convw_ref, fc0w_ref, fc0b_ref,
            h0_ref, cur1_ref, ca_ref, cw_ref,
            h0_s):
    l = pl.program_id(0)
    rb = pl.program_id(1)
    r0 = rb * BR_A

    @pl.when(l == 0)
    def _prologue():
        for b, ref in ((0, x_ref), (1, wf_ref)):
            h = jnp.maximum(ref[...] @ fc0w_ref[...] + fc0b_ref[...], 0.0)
            h0_s[b, pl.ds(r0, BR_A), :] = h
            h0_ref[b, :, :] = h

    @pl.when(l == 1)
    def _layer1():
        theta = jnp.float32(_THETAS[0])
        w = convw_ref[0]
        for b, aref, cref in ((0, adj_ref, ca_ref), (1, wadj_ref, cw_ref)):
            a = aref[...].astype(jnp.bfloat16)
            cref[...] = a
            h0_full = h0_s[b, :, :]
            h_hi, h_lo = _split_pair(h0_full)
            rhs = jnp.concatenate([h_hi, h_lo], axis=1)
            hh = jnp.dot(a, rhs, preferred_element_type=jnp.float32)
            hi = hh[:, :NHID] + hh[:, NHID:]
            h0_rows = h0_s[b, pl.ds(r0, BR_A), :]
            sup = (1.0 - ALPHA) * hi + ALPHA * h0_rows
            out = theta * jnp.dot(sup, w, preferred_element_type=jnp.float32) \
                + (1.0 - theta) * sup + h0_rows
            new = jnp.maximum(out, 0.0)
            n_hi, n_lo = _split_pair(new)
            cur1_ref[b, :, 0:NHID] = n_hi
            cur1_ref[b, :, NHID:2 * NHID] = n_lo


def _body_b(mut_ref,
            ca_ref, cw_ref, h0_ref, cur1_ref, convw_ref, fcwt_ref,
            fcb_ref, fc2w_ref, fc2b_ref, aux_ref,
            o1_ref, o2_ref,
            cur_s, fin_s):
    l = pl.program_id(0)
    rb = pl.program_id(1)
    r0 = rb * BR_B
    NL = NLAYERS - 1

    @pl.when((l == 0) & (rb == 0))
    def _seed():
        cur_s[0, :, :, :] = cur1_ref[...]

    rp = l % 2
    wp = (l + 1) % 2
    theta = jnp.where(l == 0, _THETAS[1],
            jnp.where(l == 1, _THETAS[2], _THETAS[3])).astype(jnp.float32)
    w = convw_ref[l + 1]
    for b, cref in ((0, ca_ref), (1, cw_ref)):
        a = cref[...]
        rhs = cur_s[rp, b, :, :]
        hh = jnp.dot(a, rhs, preferred_element_type=jnp.float32)
        hi = hh[:, :NHID] + hh[:, NHID:]
        sup = (1.0 - ALPHA) * hi + ALPHA * h0_ref[b, pl.ds(r0, BR_B), :]
        blk = cur_s[rp, b, pl.ds(r0, BR_B), :].astype(jnp.float32)
        inp_rows = blk[:, :NHID] + blk[:, NHID:]
        out = theta * jnp.dot(sup, w, preferred_element_type=jnp.float32) \
            + (1.0 - theta) * sup + inp_rows
        new = jnp.maximum(out, 0.0)
        n_hi, n_lo = _split_pair(new)
        cur_s[wp, b, pl.ds(r0, BR_B), 0:NHID] = n_hi
        cur_s[wp, b, pl.ds(r0, BR_B), NHID:2 * NHID] = n_lo

        @pl.when(l == NL - 1)
        def _():
            fin_s[b, pl.ds(r0, BR_B), :] = new

    @pl.when((l == NL - 1) & (rb == NRB_B - 1))
    def _head():
        sums = []
        for b in (0, 1):
            acc = jnp.zeros((1, NHID), dtype=jnp.float32)
            for i in range(M):
                acc = acc + fin_s[b, pl.ds(mut_ref[i], 1), :]
            sums.append(acc)
        differ = sums[0] - sums[1]
        lid = jnp.sum(differ * fcwt_ref[...], axis=1, keepdims=True) \
            + fcb_ref[0]
        o2_ref[...] = lid
        relu_lid = jnp.maximum(lid, 0.0)
        o1_ref[...] = relu_lid * fc2w_ref[0] \
            + (aux_ref[4] * M) * fc2w_ref[1] \
            + (aux_ref[5] * M) * fc2w_ref[2] + fc2b_ref[0]


def kernel(x, adj, wild_adj, wild_feature, nodes, mutaion_site, aux,
           fc0_w, fc0_b, conv_W, fc_w, fc_b, fc2_w, fc2_b):
    del nodes
    f32 = jnp.float32

    in_specs_a = [
        pl.BlockSpec((BR_A, NFEAT), lambda l, rb: (jnp.where(l == 0, rb, 0), 0)),
        pl.BlockSpec((BR_A, NFEAT), lambda l, rb: (jnp.where(l == 0, rb, 0), 0)),
        pl.BlockSpec((BR_A, N), lambda l, rb: (jnp.where(l == 0, 0, rb), 0)),
        pl.BlockSpec((BR_A, N), lambda l, rb: (jnp.where(l == 0, 0, rb), 0)),
        pl.BlockSpec((NLAYERS, NHID, NHID), lambda l, rb: (0, 0, 0)),
        pl.BlockSpec((NFEAT, NHID), lambda l, rb: (0, 0)),
        pl.BlockSpec((1, NHID), lambda l, rb: (0, 0)),
    ]
    out_specs_a = [
        pl.BlockSpec((2, BR_A, NHID),
                     lambda l, rb: (0, jnp.where(l == 0, rb, NRB_A - 1), 0)),
        pl.BlockSpec((2, BR_A, 2 * NHID), lambda l, rb: (0, jnp.where(l == 0, 0, rb), 0)),
        pl.BlockSpec((BR_A, N), lambda l, rb: (jnp.where(l == 0, 0, rb), 0)),
        pl.BlockSpec((BR_A, N), lambda l, rb: (jnp.where(l == 0, 0, rb), 0)),
    ]
    h0, cur1, cache_a, cache_w = pl.pallas_call(
        _body_a,
        grid=(2, NRB_A),
        in_specs=in_specs_a,
        out_specs=out_specs_a,
        out_shape=[
            jax.ShapeDtypeStruct((2, N, NHID), f32),
            jax.ShapeDtypeStruct((2, N, 2 * NHID), jnp.bfloat16),
            jax.ShapeDtypeStruct((N, N), jnp.bfloat16),
            jax.ShapeDtypeStruct((N, N), jnp.bfloat16),
        ],
        scratch_shapes=[pltpu.VMEM((2, N, NHID), f32)],
        compiler_params=pltpu.CompilerParams(
            dimension_semantics=("arbitrary", "arbitrary"),
        ),
    )(x, wild_feature, adj, wild_adj, conv_W, fc0_w, fc0_b.reshape(1, NHID))

    in_specs_b = [
        pl.BlockSpec((BR_B, N), lambda l, rb, *_: (rb, 0)),
        pl.BlockSpec((BR_B, N), lambda l, rb, *_: (rb, 0)),
        pl.BlockSpec((2, N, NHID), lambda l, rb, *_: (0, 0, 0)),
        pl.BlockSpec((2, N, 2 * NHID), lambda l, rb, *_: (0, 0, 0)),
        pl.BlockSpec((NLAYERS, NHID, NHID), lambda l, rb, *_: (0, 0, 0)),
        pl.BlockSpec((1, NHID), lambda l, rb, *_: (0, 0)),
        pl.BlockSpec(memory_space=pltpu.SMEM),
        pl.BlockSpec(memory_space=pltpu.SMEM),
        pl.BlockSpec(memory_space=pltpu.SMEM),
        pl.BlockSpec(memory_space=pltpu.SMEM),
    ]
    out_specs_b = [
        pl.BlockSpec((1, 1), lambda l, rb, *_: (0, 0)),
        pl.BlockSpec((1, 1), lambda l, rb, *_: (0, 0)),
    ]
    grid_spec_b = pltpu.PrefetchScalarGridSpec(
        num_scalar_prefetch=1,
        grid=(NLAYERS - 1, NRB_B),
        in_specs=in_specs_b,
        out_specs=out_specs_b,
        scratch_shapes=[
            pltpu.VMEM((2, 2, N, 2 * NHID), jnp.bfloat16),
            pltpu.VMEM((2, N, NHID), f32),
        ],
    )
    o1, o2 = pl.pallas_call(
        _body_b,
        grid_spec=grid_spec_b,
        out_shape=[
            jax.ShapeDtypeStruct((1, 1), f32),
            jax.ShapeDtypeStruct((1, 1), f32),
        ],
        compiler_params=pltpu.CompilerParams(
            dimension_semantics=("arbitrary", "arbitrary"),
        ),
    )(mutaion_site.astype(jnp.int32),
      cache_a, cache_w, h0, cur1, conv_W, fc_w.reshape(1, NHID),
      fc_b, fc2_w.reshape(3), fc2_b, aux)
    return (o1.reshape(1), o2.reshape(1))
